# Initial kernel scaffold; baseline (speedup 1.0000x reference)
import jax
import jax.numpy as jnp
from jax.experimental import pallas as pl


def kernel(P_Q_inj, senders, receivers, edge_features, params):
    raise NotImplementedError("write your pallas kernel here")


# speedup vs baseline: 2.7903x; 2.7903x over previous
import functools

import jax
import jax.numpy as jnp
from jax import lax
from jax.experimental import pallas as pl
from jax.experimental.pallas import tpu as pltpu
from jax.experimental.pallas import tpu_sc as plsc

NC = 2
NS = 16
LANE = 16
IDXW = 128
BK = 4


def _chunks(total, step):
    out = []
    off = 0
    while off < total:
        c = min(step, total - off)
        out.append((off, c))
        off += c
    return out


@functools.lru_cache(maxsize=None)
def _make_edge_kernel(n_nodes, hid, rows_total, interpret=False):
    half = n_nodes // 2
    trash = half
    zrows = (-(-(half + 1) // NS) + 7) // 8 * 8
    acc_rows = zrows * NS
    rows_per_tile = rows_total // NS
    nblk = rows_per_tile // BK
    orows = -(-half // NS) // 8 * 8
    orows_last = half - orows * (NS - 1)
    assert orows_last > 0 and orows_last % 8 == 0 and half % 8 == 0

    mesh = plsc.VectorSubcoreMesh(core_axis_name="c", subcore_axis_name="s",
                                  num_cores=NC, num_subcores=NS)

    @functools.partial(
        pl.kernel,
        out_type=jax.ShapeDtypeStruct((n_nodes, hid), jnp.float32),
        mesh=mesh,
        scratch_types=[
            pltpu.VMEM((BK, IDXW), jnp.int32),
            pltpu.VMEM((BK, IDXW), jnp.int32),
            pltpu.VMEM((BK * IDXW, hid), jnp.float32),
            pltpu.VMEM_SHARED((acc_rows, hid), jnp.float32),
            pltpu.SemaphoreType.DMA,
        ],
        compiler_params=pltpu.CompilerParams(use_tc_tiling_on_sc=False),
        interpret=interpret,
    )
    def edge_kernel(t_hbm, s_hbm, r_hbm, out_hbm, sidx_v, ridx_v, rows_v,
                    acc_sh, sem):
        c = lax.axis_index("c")
        s = lax.axis_index("s")
        zero16 = jnp.zeros((LANE,), jnp.float32)

        def zbody(i, carry):
            for k in range(hid // LANE):
                rows_v[i, pl.ds(k * LANE, LANE)] = zero16
            return carry

        lax.fori_loop(0, BK * IDXW, zbody, 0)
        zbase = s * zrows
        for off, cnt in _chunks(zrows, BK * IDXW):
            pltpu.sync_copy(rows_v.at[pl.ds(0, cnt)],
                            acc_sh.at[pl.ds(zbase + off, cnt)])
        plsc.subcore_barrier()

        row_base = s * rows_per_tile
        node_base = c * half

        def block(b, carry):
            r0 = row_base + b * BK
            pltpu.sync_copy(s_hbm.at[pl.ds(r0, BK)], sidx_v)
            pltpu.sync_copy(r_hbm.at[pl.ds(r0, BK)], ridx_v)
            descs = [
                pltpu.async_copy(t_hbm.at[sidx_v.at[j]],
                                 rows_v.at[pl.ds(j * IDXW, IDXW)], sem)
                for j in range(BK)
            ]
            for j in range(BK):
                for k in range(IDXW // LANE):
                    r = ridx_v[j, pl.ds(k * LANE, LANE)]
                    rl = r - node_base
                    ok = (rl >= 0) & (rl < half)
                    ridx_v[j, pl.ds(k * LANE, LANE)] = jnp.where(ok, rl, trash)
            for d in descs:
                d.wait()
            for j in range(BK):
                pltpu.sync_copy(rows_v.at[pl.ds(j * IDXW, IDXW)],
                                acc_sh.at[ridx_v.at[j]], add=True)
            return carry

        lax.fori_loop(0, nblk, block, 0)
        plsc.subcore_barrier()

        obase = s * orows

        @pl.when(s < NS - 1)
        def _():
            for off, cnt in _chunks(orows, BK * IDXW):
                pltpu.sync_copy(acc_sh.at[pl.ds(obase + off, cnt)],
                                rows_v.at[pl.ds(0, cnt)])
                pltpu.sync_copy(rows_v.at[pl.ds(0, cnt)],
                                out_hbm.at[pl.ds(node_base + obase + off, cnt)])

        @pl.when(s == NS - 1)
        def _():
            for off, cnt in _chunks(orows_last, BK * IDXW):
                pltpu.sync_copy(acc_sh.at[pl.ds(obase + off, cnt)],
                                rows_v.at[pl.ds(0, cnt)])
                pltpu.sync_copy(rows_v.at[pl.ds(0, cnt)],
                                out_hbm.at[pl.ds(node_base + obase + off, cnt)])

    return edge_kernel


def kernel(P_Q_inj, senders, receivers, edge_features, params):
    N = P_Q_inj.shape[0]
    E = senders.shape[0]
    H = params["W0"].shape[1]

    rows_min = -(-E // IDXW)
    rows_total = -(-rows_min // (NS * BK)) * (NS * BK)
    epad = rows_total * IDXW
    s2 = jnp.concatenate(
        [senders.astype(jnp.int32), jnp.zeros((epad - E,), jnp.int32)]
    ).reshape(rows_total, IDXW)
    r2 = jnp.concatenate(
        [receivers.astype(jnp.int32), jnp.full((epad - E,), N, jnp.int32)]
    ).reshape(rows_total, IDXW)

    ones = jnp.ones((E, 1), jnp.float32)
    ef1 = jnp.concatenate([edge_features, ones], axis=-1)
    efs1 = jax.ops.segment_sum(ef1, receivers, num_segments=N)
    efs, deg = efs1[:, :-1], efs1[:, -1:]

    edge_call = _make_edge_kernel(N, H, rows_total)

    V = jnp.zeros_like(P_Q_inj).at[:, 0].set(1.0)
    h = P_Q_inj @ params["W0"] + params["b0"]
    g = jnp.zeros((1, H), jnp.float32)
    for lp in params["layers"]:
        Wm = lp["Wm"]
        nin = 2 + H
        T = jnp.concatenate([V, h], axis=-1) @ Wm[:nin]
        agg = edge_call(T, s2, r2)
        agg = agg + efs @ Wm[nin:] + deg * lp["bm"][None, :]
        h = jax.nn.relu(agg)
        nm = jnp.mean(h, axis=0, keepdims=True)
        g = jnp.concatenate([g, nm], axis=-1) @ lp["Wg"] + lp["bg"]
        h = jnp.concatenate([h, jnp.broadcast_to(g, (N, H))], axis=-1) @ lp["Wn"] + lp["bn"]
        V = V + h @ lp["Wd"] + lp["bd"]
    return V


# speedup vs baseline: 4.6650x; 1.6718x over previous
import functools

import jax
import jax.numpy as jnp
from jax import lax
from jax.experimental import pallas as pl
from jax.experimental.pallas import tpu as pltpu
from jax.experimental.pallas import tpu_sc as plsc

NC = 2
NS = 16
LANE = 16
IDXW = 128
BLK = 256
KB = 3
BODY = KB * BLK


def _chunks(total, step):
    out = []
    off = 0
    while off < total:
        c = min(step, total - off)
        out.append((off, c))
        off += c
    return out


def _mesh():
    return plsc.VectorSubcoreMesh(core_axis_name="c", subcore_axis_name="s",
                                  num_cores=NC, num_subcores=NS)


_CPARAMS = pltpu.CompilerParams(use_tc_tiling_on_sc=False)


def _plan(n_nodes, n_edges):
    half = n_nodes // 2
    trash = half
    zrows = (-(-(half + 1) // NS) + 7) // 8 * 8
    acc_rows = zrows * NS
    e_tile = n_edges // NS
    nbody = e_tile // BODY
    rem = e_tile - nbody * BODY
    orows = -(-half // NS) // 8 * 8
    orows_last = half - orows * (NS - 1)
    assert rem % LANE == 0 and orows_last > 0 and orows_last % 8 == 0
    assert e_tile % 8 == 0 and n_edges % NS == 0
    return half, trash, zrows, acc_rows, e_tile, nbody, rem, orows, orows_last


def _remap_block(rbuf, ridx2, n_valid, node_base, half, trash):
    trash_v = jnp.full((LANE,), trash, jnp.int32)
    for i in range(BODY // LANE):
        row, col = divmod(i * LANE, IDXW)
        if i * LANE >= n_valid:
            ridx2[row, pl.ds(col, LANE)] = trash_v
        else:
            r = rbuf[pl.ds(i * LANE, LANE)]
            rl = r - node_base
            ok = (rl >= 0) & (rl < half)
            ridx2[row, pl.ds(col, LANE)] = jnp.where(ok, rl, trash)


@functools.lru_cache(maxsize=None)
def _make_edge_kernel(n_nodes, n_edges, hid, table_rows):
    (half, trash, zrows, acc_rows, e_tile, nbody, rem,
     orows, orows_last) = _plan(n_nodes, n_edges)
    del table_rows

    @functools.partial(
        pl.kernel,
        out_type=jax.ShapeDtypeStruct((n_nodes, hid), jnp.float32),
        mesh=_mesh(),
        scratch_types=[
            pltpu.VMEM((BODY,), jnp.int32),
            pltpu.VMEM((BODY,), jnp.int32),
            pltpu.VMEM((BODY // IDXW, IDXW), jnp.int32),
            [pltpu.VMEM((BLK, hid), jnp.float32) for _ in range(KB)],
            pltpu.VMEM_SHARED((acc_rows, hid), jnp.float32),
            pltpu.SemaphoreType.DMA,
            pltpu.SemaphoreType.DMA,
        ],
        compiler_params=_CPARAMS,
    )
    def edge_kernel(t_hbm, s_hbm, r_hbm, z_hbm, out_hbm, sbuf, rbuf, ridx2,
                    rows, acc_sh, gsem, ssem):
        c = lax.axis_index("c")
        s = lax.axis_index("s")
        node_base = c * half

        pltpu.sync_copy(z_hbm, rows[0])
        zbase = s * zrows
        for off, cnt in _chunks(zrows, BLK):
            pltpu.sync_copy(rows[0].at[pl.ds(0, cnt)],
                            acc_sh.at[pl.ds(zbase + off, cnt)])
        plsc.subcore_barrier()

        ebase = s * e_tile

        def run_body(e0, n_valid):
            nv8 = -(-n_valid // 8) * 8
            i1 = pltpu.async_copy(s_hbm.at[pl.ds(e0, nv8)],
                                  sbuf.at[pl.ds(0, nv8)], gsem)
            i2 = pltpu.async_copy(r_hbm.at[pl.ds(e0, nv8)],
                                  rbuf.at[pl.ds(0, nv8)], gsem)
            i1.wait()
            i2.wait()
            if n_valid < BODY:
                zero_i = jnp.zeros((LANE,), jnp.int32)
                for i in range(n_valid // LANE, BODY // LANE):
                    sbuf[pl.ds(i * LANE, LANE)] = zero_i
            gds = []
            for k in range(KB):
                for j in range(BLK // IDXW):
                    gds.append(pltpu.async_copy(
                        t_hbm.at[sbuf.at[pl.ds(k * BLK + j * IDXW, IDXW)]],
                        rows[k].at[pl.ds(j * IDXW, IDXW)], gsem))
            _remap_block(rbuf, ridx2, n_valid, node_base, half, trash)
            sds = []
            for k in range(KB):
                for j in range(BLK // IDXW):
                    gds[k * (BLK // IDXW) + j].wait()
                    sds.append(pltpu.async_copy(
                        rows[k].at[pl.ds(j * IDXW, IDXW)],
                        acc_sh.at[ridx2.at[k * (BLK // IDXW) + j]],
                        ssem, add=True))
            for d in sds:
                d.wait()

        def body(b, carry):
            run_body(ebase + b * BODY, BODY)
            return carry

        lax.fori_loop(0, nbody, body, 0)
        if rem:
            run_body(ebase + nbody * BODY, rem)
        plsc.subcore_barrier()

        def writeout(n_out):
            obase = s * orows
            for off, cnt in _chunks(n_out, BLK):
                pltpu.sync_copy(acc_sh.at[pl.ds(obase + off, cnt)],
                                rows[0].at[pl.ds(0, cnt)])
                pltpu.sync_copy(rows[0].at[pl.ds(0, cnt)],
                                out_hbm.at[pl.ds(node_base + obase + off, cnt)])

        @pl.when(s < NS - 1)
        def _():
            writeout(orows)

        @pl.when(s == NS - 1)
        def _():
            writeout(orows_last)

    return edge_kernel


def kernel(P_Q_inj, senders, receivers, edge_features, params):
    N = P_Q_inj.shape[0]
    E = senders.shape[0]
    H = params["W0"].shape[1]
    D = edge_features.shape[1]

    s1 = senders.astype(jnp.int32)
    r1 = receivers.astype(jnp.int32)

    zeros_h = jnp.zeros((BLK, H), jnp.float32)

    edge_call = _make_edge_kernel(N, E, H, N)
    pre_call = _make_edge_kernel(N, E, H, E)

    ef32 = jnp.concatenate(
        [edge_features, jnp.ones((E, 1), jnp.float32),
         jnp.zeros((E, H - D - 1), jnp.float32)], axis=-1)
    eidx = jnp.arange(E, dtype=jnp.int32)
    pre = pre_call(ef32, eidx, r1, zeros_h)
    efs, deg = pre[:, :D], pre[:, D:D + 1]

    V = jnp.zeros_like(P_Q_inj).at[:, 0].set(1.0)
    h = P_Q_inj @ params["W0"] + params["b0"]
    g = jnp.zeros((1, H), jnp.float32)
    for lp in params["layers"]:
        Wm = lp["Wm"]
        nin = 2 + H
        T = jnp.concatenate([V, h], axis=-1) @ Wm[:nin]
        agg = edge_call(T, s1, r1, zeros_h)
        agg = agg + efs @ Wm[nin:] + deg * lp["bm"][None, :]
        h = jax.nn.relu(agg)
        nm = jnp.mean(h, axis=0, keepdims=True)
        g = jnp.concatenate([g, nm], axis=-1) @ lp["Wg"] + lp["bg"]
        h = jnp.concatenate([h, jnp.broadcast_to(g, (N, H))], axis=-1) @ lp["Wn"] + lp["bn"]
        V = V + h @ lp["Wd"] + lp["bd"]
    return V


# speedup vs baseline: 5.1111x; 1.0956x over previous
import functools

import jax
import jax.numpy as jnp
from jax import lax
from jax.experimental import pallas as pl
from jax.experimental.pallas import tpu as pltpu
from jax.experimental.pallas import tpu_sc as plsc

NC = 2
NS = 16
LANE = 16
IDXW = 128
NR = 3
BODY = NR * IDXW


def _chunks(total, step):
    out = []
    off = 0
    while off < total:
        c = min(step, total - off)
        out.append((off, c))
        off += c
    return out


def _mesh():
    return plsc.VectorSubcoreMesh(core_axis_name="c", subcore_axis_name="s",
                                  num_cores=NC, num_subcores=NS)


_CPARAMS = pltpu.CompilerParams(use_tc_tiling_on_sc=False)


def _plan(n_nodes, n_edges):
    half = n_nodes // 2
    trash = half
    zrows = (-(-(half + 1) // NS) + 7) // 8 * 8
    acc_rows = zrows * NS
    e_tile = n_edges // NS
    npair = e_tile // (2 * BODY)
    rem = e_tile - npair * 2 * BODY
    orows = -(-half // NS) // 8 * 8
    orows_last = half - orows * (NS - 1)
    assert rem % LANE == 0 and orows_last > 0 and orows_last % 8 == 0
    assert e_tile % 8 == 0 and n_edges % NS == 0
    return half, trash, zrows, acc_rows, e_tile, npair, rem, orows, orows_last


@functools.lru_cache(maxsize=None)
def _make_edge_kernel(n_nodes, n_edges, hid, table_rows):
    (half, trash, zrows, acc_rows, e_tile, npair, rem,
     orows, orows_last) = _plan(n_nodes, n_edges)
    del table_rows

    @functools.partial(
        pl.kernel,
        out_type=jax.ShapeDtypeStruct((n_nodes, hid), jnp.float32),
        mesh=_mesh(),
        scratch_types=[
            [pltpu.VMEM((BODY,), jnp.int32) for _ in range(2)],
            [pltpu.VMEM((BODY,), jnp.int32) for _ in range(2)],
            [pltpu.VMEM((BODY, hid), jnp.float32) for _ in range(2)],
            pltpu.VMEM_SHARED((acc_rows, hid), jnp.float32),
            pltpu.SemaphoreType.DMA,
            pltpu.SemaphoreType.DMA,
            pltpu.SemaphoreType.DMA,
        ],
        compiler_params=_CPARAMS,
    )
    def edge_kernel(t_hbm, s_hbm, r_hbm, z_hbm, out_hbm, sbuf, rbuf,
                    rows, acc_sh, isem, gsem, ssem):
        c = lax.axis_index("c")
        s = lax.axis_index("s")
        node_base = c * half

        pltpu.sync_copy(z_hbm, rows[0])
        zbase = s * zrows
        for off, cnt in _chunks(zrows, BODY):
            pltpu.sync_copy(rows[0].at[pl.ds(0, cnt)],
                            acc_sh.at[pl.ds(zbase + off, cnt)])
        plsc.subcore_barrier()

        ebase = s * e_tile

        def load_idx(e0, n_valid, p):
            nv8 = -(-n_valid // 8) * 8
            return (pltpu.async_copy(s_hbm.at[pl.ds(e0, nv8)],
                                     sbuf[p].at[pl.ds(0, nv8)], isem),
                    pltpu.async_copy(r_hbm.at[pl.ds(e0, nv8)],
                                     rbuf[p].at[pl.ds(0, nv8)], isem))

        def prep(n_valid, p):
            trash_v = jnp.full((LANE,), trash, jnp.int32)
            zero_v = jnp.zeros((LANE,), jnp.int32)
            for i in range(BODY // LANE):
                if i * LANE >= n_valid:
                    sbuf[p][pl.ds(i * LANE, LANE)] = zero_v
                    rbuf[p][pl.ds(i * LANE, LANE)] = trash_v
                else:
                    r = rbuf[p][pl.ds(i * LANE, LANE)]
                    rl = r - node_base
                    ok = (rl >= 0) & (rl < half)
                    rbuf[p][pl.ds(i * LANE, LANE)] = jnp.where(ok, rl, trash)

        def fire_gather(p):
            return pltpu.async_copy(t_hbm.at[sbuf[p]], rows[p], gsem)

        def fire_scatter(p):
            return pltpu.async_copy(rows[p], acc_sh.at[rbuf[p]], ssem,
                                    add=True)

        def pair(e0):
            iA = load_idx(e0, BODY, 0)
            iB = load_idx(e0 + BODY, BODY, 1)
            iA[0].wait()
            iA[1].wait()
            prep(BODY, 0)
            gA = fire_gather(0)
            iB[0].wait()
            iB[1].wait()
            prep(BODY, 1)
            gA.wait()
            sA = fire_scatter(0)
            gB = fire_gather(1)
            gB.wait()
            sA.wait()
            sB = fire_scatter(1)
            sB.wait()

        def body(b, carry):
            pair(ebase + b * 2 * BODY)
            return carry

        lax.fori_loop(0, npair, body, 0)
        for off, cnt in _chunks(rem, BODY):
            i0 = load_idx(ebase + npair * 2 * BODY + off, cnt, 0)
            i0[0].wait()
            i0[1].wait()
            prep(cnt, 0)
            fire_gather(0).wait()
            fire_scatter(0).wait()
        plsc.subcore_barrier()

        def writeout(n_out):
            obase = s * orows
            for off, cnt in _chunks(n_out, BODY):
                pltpu.sync_copy(acc_sh.at[pl.ds(obase + off, cnt)],
                                rows[0].at[pl.ds(0, cnt)])
                pltpu.sync_copy(rows[0].at[pl.ds(0, cnt)],
                                out_hbm.at[pl.ds(node_base + obase + off, cnt)])

        @pl.when(s < NS - 1)
        def _():
            writeout(orows)

        @pl.when(s == NS - 1)
        def _():
            writeout(orows_last)

    return edge_kernel


def kernel(P_Q_inj, senders, receivers, edge_features, params):
    N = P_Q_inj.shape[0]
    E = senders.shape[0]
    H = params["W0"].shape[1]
    D = edge_features.shape[1]

    s1 = senders.astype(jnp.int32)
    r1 = receivers.astype(jnp.int32)

    zeros_h = jnp.zeros((BODY, H), jnp.float32)

    edge_call = _make_edge_kernel(N, E, H, N)
    pre_call = _make_edge_kernel(N, E, H, E)

    ef32 = jnp.concatenate(
        [edge_features, jnp.ones((E, 1), jnp.float32),
         jnp.zeros((E, H - D - 1), jnp.float32)], axis=-1)
    eidx = jnp.arange(E, dtype=jnp.int32)
    pre = pre_call(ef32, eidx, r1, zeros_h)
    efs, deg = pre[:, :D], pre[:, D:D + 1]

    V = jnp.zeros_like(P_Q_inj).at[:, 0].set(1.0)
    h = P_Q_inj @ params["W0"] + params["b0"]
    g = jnp.zeros((1, H), jnp.float32)
    for lp in params["layers"]:
        Wm = lp["Wm"]
        nin = 2 + H
        T = jnp.concatenate([V, h], axis=-1) @ Wm[:nin]
        agg = edge_call(T, s1, r1, zeros_h)
        agg = agg + efs @ Wm[nin:] + deg * lp["bm"][None, :]
        h = jax.nn.relu(agg)
        nm = jnp.mean(h, axis=0, keepdims=True)
        g = jnp.concatenate([g, nm], axis=-1) @ lp["Wg"] + lp["bg"]
        h = jnp.concatenate([h, jnp.broadcast_to(g, (N, H))], axis=-1) @ lp["Wn"] + lp["bn"]
        V = V + h @ lp["Wd"] + lp["bd"]
    return V


# speedup vs baseline: 8.2510x; 1.6143x over previous
import functools

import jax
import jax.numpy as jnp
from jax import lax
from jax.experimental import pallas as pl
from jax.experimental.pallas import tpu as pltpu
from jax.experimental.pallas import tpu_sc as plsc

NC = 2
NS = 16
LANE = 16
IDXW = 128
NR = 3
BODY = NR * IDXW


def _chunks(total, step):
    out = []
    off = 0
    while off < total:
        c = min(step, total - off)
        out.append((off, c))
        off += c
    return out


def _mesh():
    return plsc.VectorSubcoreMesh(core_axis_name="c", subcore_axis_name="s",
                                  num_cores=NC, num_subcores=NS)


_CPARAMS = pltpu.CompilerParams(use_tc_tiling_on_sc=False)


def _plan(n_nodes, n_edges):
    half = n_nodes // 2
    trash = half
    zrows = (-(-(half + 1) // NS) + 7) // 8 * 8
    acc_rows = zrows * NS
    e_tile = n_edges // NS
    npair = e_tile // (2 * BODY)
    rem = e_tile - npair * 2 * BODY
    orows = -(-half // NS) // 8 * 8
    orows_last = half - orows * (NS - 1)
    assert rem % LANE == 0 and orows_last > 0 and orows_last % 8 == 0
    assert e_tile % 8 == 0 and n_edges % NS == 0
    return half, trash, zrows, acc_rows, e_tile, npair, rem, orows, orows_last


@functools.lru_cache(maxsize=None)
def _make_edge_kernel(n_nodes, n_edges, hid, table_rows):
    (half, trash, zrows, acc_rows, e_tile, npair, rem,
     orows, orows_last) = _plan(n_nodes, n_edges)
    del table_rows

    @functools.partial(
        pl.kernel,
        out_type=jax.ShapeDtypeStruct((n_nodes, hid), jnp.float32),
        mesh=_mesh(),
        scratch_types=[
            [pltpu.VMEM((BODY,), jnp.int32) for _ in range(2)],
            [pltpu.VMEM((BODY,), jnp.int32) for _ in range(2)],
            [pltpu.VMEM((BODY, hid), jnp.float32) for _ in range(2)],
            pltpu.VMEM_SHARED((acc_rows, hid), jnp.float32),
            pltpu.SemaphoreType.DMA,
            pltpu.SemaphoreType.DMA,
            pltpu.SemaphoreType.DMA,
        ],
        compiler_params=_CPARAMS,
    )
    def edge_kernel(t_hbm, s_hbm, r_hbm, z_hbm, out_hbm, sbuf, rbuf,
                    rows, acc_sh, isem, gsem, ssem):
        c = lax.axis_index("c")
        s = lax.axis_index("s")
        node_base = c * half

        pltpu.sync_copy(z_hbm, rows[0])
        zbase = s * zrows
        for off, cnt in _chunks(zrows, BODY):
            pltpu.sync_copy(rows[0].at[pl.ds(0, cnt)],
                            acc_sh.at[pl.ds(zbase + off, cnt)])
        plsc.subcore_barrier()

        ebase = s * e_tile

        def load_idx(e0, n_valid, p):
            nv8 = -(-n_valid // 8) * 8
            return (pltpu.async_copy(s_hbm.at[pl.ds(e0, nv8)],
                                     sbuf[p].at[pl.ds(0, nv8)], isem),
                    pltpu.async_copy(r_hbm.at[pl.ds(e0, nv8)],
                                     rbuf[p].at[pl.ds(0, nv8)], isem))

        def prep(n_valid, p):
            trash_v = jnp.full((LANE,), trash, jnp.int32)
            zero_v = jnp.zeros((LANE,), jnp.int32)
            for i in range(BODY // LANE):
                if i * LANE >= n_valid:
                    sbuf[p][pl.ds(i * LANE, LANE)] = zero_v
                    rbuf[p][pl.ds(i * LANE, LANE)] = trash_v
                else:
                    r = rbuf[p][pl.ds(i * LANE, LANE)]
                    rl = r - node_base
                    ok = (rl >= 0) & (rl < half)
                    rbuf[p][pl.ds(i * LANE, LANE)] = jnp.where(ok, rl, trash)

        def fire_gather(p):
            return pltpu.async_copy(t_hbm.at[sbuf[p]], rows[p], gsem)

        def fire_scatter(p):
            return pltpu.async_copy(rows[p], acc_sh.at[rbuf[p]], ssem,
                                    add=True)

        def pair(e0):
            iA = load_idx(e0, BODY, 0)
            iB = load_idx(e0 + BODY, BODY, 1)
            iA[0].wait()
            iA[1].wait()
            prep(BODY, 0)
            gA = fire_gather(0)
            iB[0].wait()
            iB[1].wait()
            prep(BODY, 1)
            gA.wait()
            sA = fire_scatter(0)
            gB = fire_gather(1)
            gB.wait()
            sA.wait()
            sB = fire_scatter(1)
            sB.wait()

        def body(b, carry):
            pair(ebase + b * 2 * BODY)
            return carry

        lax.fori_loop(0, npair, body, 0)
        for off, cnt in _chunks(rem, BODY):
            i0 = load_idx(ebase + npair * 2 * BODY + off, cnt, 0)
            i0[0].wait()
            i0[1].wait()
            prep(cnt, 0)
            fire_gather(0).wait()
            fire_scatter(0).wait()
        plsc.subcore_barrier()

        def writeout(n_out):
            obase = s * orows
            for off, cnt in _chunks(n_out, BODY):
                pltpu.sync_copy(acc_sh.at[pl.ds(obase + off, cnt)],
                                rows[0].at[pl.ds(0, cnt)])
                pltpu.sync_copy(rows[0].at[pl.ds(0, cnt)],
                                out_hbm.at[pl.ds(node_base + obase + off, cnt)])

        @pl.when(s < NS - 1)
        def _():
            writeout(orows)

        @pl.when(s == NS - 1)
        def _():
            writeout(orows_last)

    return edge_kernel


@functools.lru_cache(maxsize=None)
def _make_edge_kernel_bf16(n_nodes, n_edges, hid):
    trash = n_nodes
    zrows = (-(-(n_nodes + 1) // NS) + 7) // 8 * 8
    acc_rows = zrows * NS
    e_w = n_edges // (NC * NS)
    npair = e_w // (2 * BODY)
    rem = e_w - npair * 2 * BODY
    orows = -(-n_nodes // NS) // 8 * 8
    orows_last = n_nodes - orows * (NS - 1)
    assert rem % LANE == 0 and orows_last > 0 and orows_last % 8 == 0
    assert e_w % 8 == 0 and n_edges % (NC * NS) == 0

    @functools.partial(
        pl.kernel,
        out_type=jax.ShapeDtypeStruct((NC, n_nodes, hid), jnp.bfloat16),
        mesh=_mesh(),
        scratch_types=[
            [pltpu.VMEM((BODY,), jnp.int32) for _ in range(2)],
            [pltpu.VMEM((BODY,), jnp.int32) for _ in range(2)],
            [pltpu.VMEM((BODY, hid), jnp.bfloat16) for _ in range(2)],
            pltpu.VMEM_SHARED((acc_rows, hid), jnp.bfloat16),
            pltpu.SemaphoreType.DMA,
            pltpu.SemaphoreType.DMA,
            pltpu.SemaphoreType.DMA,
        ],
        compiler_params=_CPARAMS,
    )
    def edge_kernel(t_hbm, s_hbm, r_hbm, z_hbm, out_hbm, sbuf, rbuf,
                    rows, acc_sh, isem, gsem, ssem):
        c = lax.axis_index("c")
        s = lax.axis_index("s")

        pltpu.sync_copy(z_hbm, rows[0])
        zbase = s * zrows
        for off, cnt in _chunks(zrows, BODY):
            pltpu.sync_copy(rows[0].at[pl.ds(0, cnt)],
                            acc_sh.at[pl.ds(zbase + off, cnt)])
        plsc.subcore_barrier()

        ebase = (c * NS + s) * e_w

        def load_idx(e0, n_valid, p):
            nv8 = -(-n_valid // 8) * 8
            return (pltpu.async_copy(s_hbm.at[pl.ds(e0, nv8)],
                                     sbuf[p].at[pl.ds(0, nv8)], isem),
                    pltpu.async_copy(r_hbm.at[pl.ds(e0, nv8)],
                                     rbuf[p].at[pl.ds(0, nv8)], isem))

        def pad(n_valid, p):
            trash_v = jnp.full((LANE,), trash, jnp.int32)
            zero_v = jnp.zeros((LANE,), jnp.int32)
            for i in range(n_valid // LANE, BODY // LANE):
                sbuf[p][pl.ds(i * LANE, LANE)] = zero_v
                rbuf[p][pl.ds(i * LANE, LANE)] = trash_v

        def fire_gather(p):
            return pltpu.async_copy(t_hbm.at[sbuf[p]], rows[p], gsem)

        def fire_scatter(p):
            return pltpu.async_copy(rows[p], acc_sh.at[rbuf[p]], ssem,
                                    add=True)

        def pair(e0):
            iA = load_idx(e0, BODY, 0)
            iB = load_idx(e0 + BODY, BODY, 1)
            iA[0].wait()
            iA[1].wait()
            gA = fire_gather(0)
            iB[0].wait()
            iB[1].wait()
            gA.wait()
            sA = fire_scatter(0)
            gB = fire_gather(1)
            gB.wait()
            sA.wait()
            sB = fire_scatter(1)
            sB.wait()

        def body(b, carry):
            pair(ebase + b * 2 * BODY)
            return carry

        lax.fori_loop(0, npair, body, 0)
        for off, cnt in _chunks(rem, BODY):
            i0 = load_idx(ebase + npair * 2 * BODY + off, cnt, 0)
            i0[0].wait()
            i0[1].wait()
            pad(cnt, 0)
            fire_gather(0).wait()
            fire_scatter(0).wait()
        plsc.subcore_barrier()

        def writeout(n_out):
            obase = s * orows
            for off, cnt in _chunks(n_out, BODY):
                pltpu.sync_copy(acc_sh.at[pl.ds(obase + off, cnt)],
                                rows[0].at[pl.ds(0, cnt)])
                pltpu.sync_copy(rows[0].at[pl.ds(0, cnt)],
                                out_hbm.at[c].at[pl.ds(obase + off, cnt)])

        @pl.when(s < NS - 1)
        def _():
            writeout(orows)

        @pl.when(s == NS - 1)
        def _():
            writeout(orows_last)

    return edge_kernel


def kernel(P_Q_inj, senders, receivers, edge_features, params):
    N = P_Q_inj.shape[0]
    E = senders.shape[0]
    H = params["W0"].shape[1]
    D = edge_features.shape[1]

    s1 = senders.astype(jnp.int32)
    r1 = receivers.astype(jnp.int32)

    zeros_h = jnp.zeros((BODY, H), jnp.float32)
    zeros_hb = jnp.zeros((BODY, H), jnp.bfloat16)

    edge_call = _make_edge_kernel_bf16(N, E, H)
    pre_call = _make_edge_kernel(N, E, H, E)

    ef32 = jnp.concatenate(
        [edge_features, jnp.ones((E, 1), jnp.float32),
         jnp.zeros((E, H - D - 1), jnp.float32)], axis=-1)
    eidx = jnp.arange(E, dtype=jnp.int32)
    pre = pre_call(ef32, eidx, r1, zeros_h)
    efs, deg = pre[:, :D], pre[:, D:D + 1]

    V = jnp.zeros_like(P_Q_inj).at[:, 0].set(1.0)
    h = P_Q_inj @ params["W0"] + params["b0"]
    g = jnp.zeros((1, H), jnp.float32)
    for lp in params["layers"]:
        Wm = lp["Wm"]
        nin = 2 + H
        T = jnp.concatenate([V, h], axis=-1) @ Wm[:nin]
        parts = edge_call(T.astype(jnp.bfloat16), s1, r1, zeros_hb)
        agg = parts[0].astype(jnp.float32) + parts[1].astype(jnp.float32)
        agg = agg + efs @ Wm[nin:] + deg * lp["bm"][None, :]
        h = jax.nn.relu(agg)
        nm = jnp.mean(h, axis=0, keepdims=True)
        g = jnp.concatenate([g, nm], axis=-1) @ lp["Wg"] + lp["bg"]
        h = jnp.concatenate([h, jnp.broadcast_to(g, (N, H))], axis=-1) @ lp["Wn"] + lp["bn"]
        V = V + h @ lp["Wd"] + lp["bd"]
    return V


# speedup vs baseline: 9.6241x; 1.1664x over previous
import functools

import jax
import jax.numpy as jnp
from jax import lax
from jax.experimental import pallas as pl
from jax.experimental.pallas import tpu as pltpu
from jax.experimental.pallas import tpu_sc as plsc

NC = 2
NS = 16
LANE = 16
IDXW = 128
NR = 3
BODY = NR * IDXW


def _chunks(total, step):
    out = []
    off = 0
    while off < total:
        c = min(step, total - off)
        out.append((off, c))
        off += c
    return out


def _mesh():
    return plsc.VectorSubcoreMesh(core_axis_name="c", subcore_axis_name="s",
                                  num_cores=NC, num_subcores=NS)


_CPARAMS = pltpu.CompilerParams(use_tc_tiling_on_sc=False)


def _plan(n_nodes, n_edges):
    half = n_nodes // 2
    trash = half
    zrows = (-(-(half + 1) // NS) + 7) // 8 * 8
    acc_rows = zrows * NS
    e_tile = n_edges // NS
    npair = e_tile // (2 * BODY)
    rem = e_tile - npair * 2 * BODY
    orows = -(-half // NS) // 8 * 8
    orows_last = half - orows * (NS - 1)
    assert rem % LANE == 0 and orows_last > 0 and orows_last % 8 == 0
    assert e_tile % 8 == 0 and n_edges % NS == 0
    return half, trash, zrows, acc_rows, e_tile, npair, rem, orows, orows_last


@functools.lru_cache(maxsize=None)
def _make_edge_kernel(n_nodes, n_edges, hid, table_rows):
    (half, trash, zrows, acc_rows, e_tile, npair, rem,
     orows, orows_last) = _plan(n_nodes, n_edges)
    del table_rows

    @functools.partial(
        pl.kernel,
        out_type=jax.ShapeDtypeStruct((n_nodes, hid), jnp.float32),
        mesh=_mesh(),
        scratch_types=[
            [pltpu.VMEM((BODY,), jnp.int32) for _ in range(2)],
            [pltpu.VMEM((BODY,), jnp.int32) for _ in range(2)],
            [pltpu.VMEM((BODY, hid), jnp.float32) for _ in range(2)],
            pltpu.VMEM_SHARED((acc_rows, hid), jnp.float32),
            pltpu.SemaphoreType.DMA,
            pltpu.SemaphoreType.DMA,
            pltpu.SemaphoreType.DMA,
        ],
        compiler_params=_CPARAMS,
    )
    def edge_kernel(t_hbm, s_hbm, r_hbm, z_hbm, out_hbm, sbuf, rbuf,
                    rows, acc_sh, isem, gsem, ssem):
        c = lax.axis_index("c")
        s = lax.axis_index("s")
        node_base = c * half

        pltpu.sync_copy(z_hbm, rows[0])
        zbase = s * zrows
        for off, cnt in _chunks(zrows, BODY):
            pltpu.sync_copy(rows[0].at[pl.ds(0, cnt)],
                            acc_sh.at[pl.ds(zbase + off, cnt)])
        plsc.subcore_barrier()

        ebase = s * e_tile

        def load_idx(e0, n_valid, p):
            nv8 = -(-n_valid // 8) * 8
            return (pltpu.async_copy(s_hbm.at[pl.ds(e0, nv8)],
                                     sbuf[p].at[pl.ds(0, nv8)], isem),
                    pltpu.async_copy(r_hbm.at[pl.ds(e0, nv8)],
                                     rbuf[p].at[pl.ds(0, nv8)], isem))

        def prep(n_valid, p):
            trash_v = jnp.full((LANE,), trash, jnp.int32)
            zero_v = jnp.zeros((LANE,), jnp.int32)
            for i in range(BODY // LANE):
                if i * LANE >= n_valid:
                    sbuf[p][pl.ds(i * LANE, LANE)] = zero_v
                    rbuf[p][pl.ds(i * LANE, LANE)] = trash_v
                else:
                    r = rbuf[p][pl.ds(i * LANE, LANE)]
                    rl = r - node_base
                    ok = (rl >= 0) & (rl < half)
                    rbuf[p][pl.ds(i * LANE, LANE)] = jnp.where(ok, rl, trash)

        def fire_gather(p):
            return pltpu.async_copy(t_hbm.at[sbuf[p]], rows[p], gsem)

        def fire_scatter(p):
            return pltpu.async_copy(rows[p], acc_sh.at[rbuf[p]], ssem,
                                    add=True)

        def pair(e0):
            iA = load_idx(e0, BODY, 0)
            iB = load_idx(e0 + BODY, BODY, 1)
            iA[0].wait()
            iA[1].wait()
            prep(BODY, 0)
            gA = fire_gather(0)
            iB[0].wait()
            iB[1].wait()
            prep(BODY, 1)
            gA.wait()
            sA = fire_scatter(0)
            gB = fire_gather(1)
            gB.wait()
            sA.wait()
            sB = fire_scatter(1)
            sB.wait()

        def body(b, carry):
            pair(ebase + b * 2 * BODY)
            return carry

        lax.fori_loop(0, npair, body, 0)
        for off, cnt in _chunks(rem, BODY):
            i0 = load_idx(ebase + npair * 2 * BODY + off, cnt, 0)
            i0[0].wait()
            i0[1].wait()
            prep(cnt, 0)
            fire_gather(0).wait()
            fire_scatter(0).wait()
        plsc.subcore_barrier()

        def writeout(n_out):
            obase = s * orows
            for off, cnt in _chunks(n_out, BODY):
                pltpu.sync_copy(acc_sh.at[pl.ds(obase + off, cnt)],
                                rows[0].at[pl.ds(0, cnt)])
                pltpu.sync_copy(rows[0].at[pl.ds(0, cnt)],
                                out_hbm.at[pl.ds(node_base + obase + off, cnt)])

        @pl.when(s < NS - 1)
        def _():
            writeout(orows)

        @pl.when(s == NS - 1)
        def _():
            writeout(orows_last)

    return edge_kernel


@functools.lru_cache(maxsize=None)
def _make_edge_kernel_bf16(n_nodes, n_edges, hid):
    trash = n_nodes
    zrows = (-(-(n_nodes + 1) // NS) + 7) // 8 * 8
    acc_rows = zrows * NS
    e_w = n_edges // (NC * NS)
    npair = e_w // (2 * BODY)
    rem = e_w - npair * 2 * BODY
    orows = -(-n_nodes // NS) // 8 * 8
    orows_last = n_nodes - orows * (NS - 1)
    assert rem % LANE == 0 and orows_last > 0 and orows_last % 8 == 0
    assert e_w % 8 == 0 and n_edges % (NC * NS) == 0

    @functools.partial(
        pl.kernel,
        out_type=jax.ShapeDtypeStruct((NC, n_nodes, hid), jnp.bfloat16),
        mesh=_mesh(),
        scratch_types=[
            [pltpu.VMEM((BODY,), jnp.int32) for _ in range(2)],
            [pltpu.VMEM((BODY,), jnp.int32) for _ in range(2)],
            [pltpu.VMEM((BODY, hid), jnp.bfloat16) for _ in range(2)],
            pltpu.VMEM_SHARED((acc_rows, hid), jnp.bfloat16),
            pltpu.SemaphoreType.DMA,
            pltpu.SemaphoreType.DMA,
            pltpu.SemaphoreType.DMA,
        ],
        compiler_params=_CPARAMS,
    )
    def edge_kernel(t_hbm, s_hbm, r_hbm, z_hbm, out_hbm, sbuf, rbuf,
                    rows, acc_sh, isem, gsem, ssem):
        c = lax.axis_index("c")
        s = lax.axis_index("s")

        pltpu.sync_copy(z_hbm, rows[0])
        zbase = s * zrows
        for off, cnt in _chunks(zrows, BODY):
            pltpu.sync_copy(rows[0].at[pl.ds(0, cnt)],
                            acc_sh.at[pl.ds(zbase + off, cnt)])
        plsc.subcore_barrier()

        ebase = (c * NS + s) * e_w

        def load_idx(e0, n_valid, p):
            nv8 = -(-n_valid // 8) * 8
            return (pltpu.async_copy(s_hbm.at[pl.ds(e0, nv8)],
                                     sbuf[p].at[pl.ds(0, nv8)], isem),
                    pltpu.async_copy(r_hbm.at[pl.ds(e0, nv8)],
                                     rbuf[p].at[pl.ds(0, nv8)], isem))

        def pad(n_valid, p):
            trash_v = jnp.full((LANE,), trash, jnp.int32)
            zero_v = jnp.zeros((LANE,), jnp.int32)
            for i in range(n_valid // LANE, BODY // LANE):
                sbuf[p][pl.ds(i * LANE, LANE)] = zero_v
                rbuf[p][pl.ds(i * LANE, LANE)] = trash_v

        def fire_gather(p):
            return pltpu.async_copy(t_hbm.at[sbuf[p]], rows[p], gsem)

        def fire_scatter(p):
            return pltpu.async_copy(rows[p], acc_sh.at[rbuf[p]], ssem,
                                    add=True)

        def pair(e0):
            iA = load_idx(e0, BODY, 0)
            iB = load_idx(e0 + BODY, BODY, 1)
            iA[0].wait()
            iA[1].wait()
            gA = fire_gather(0)
            iB[0].wait()
            iB[1].wait()
            gA.wait()
            sA = fire_scatter(0)
            gB = fire_gather(1)
            gB.wait()
            sA.wait()
            sB = fire_scatter(1)
            sB.wait()

        def body(b, carry):
            pair(ebase + b * 2 * BODY)
            return carry

        lax.fori_loop(0, npair, body, 0)
        for off, cnt in _chunks(rem, BODY):
            i0 = load_idx(ebase + npair * 2 * BODY + off, cnt, 0)
            i0[0].wait()
            i0[1].wait()
            pad(cnt, 0)
            fire_gather(0).wait()
            fire_scatter(0).wait()
        plsc.subcore_barrier()

        def writeout(n_out):
            obase = s * orows
            for off, cnt in _chunks(n_out, BODY):
                pltpu.sync_copy(acc_sh.at[pl.ds(obase + off, cnt)],
                                rows[0].at[pl.ds(0, cnt)])
                pltpu.sync_copy(rows[0].at[pl.ds(0, cnt)],
                                out_hbm.at[c].at[pl.ds(obase + off, cnt)])

        @pl.when(s < NS - 1)
        def _():
            writeout(orows)

        @pl.when(s == NS - 1)
        def _():
            writeout(orows_last)

    return edge_kernel


def kernel(P_Q_inj, senders, receivers, edge_features, params):
    N = P_Q_inj.shape[0]
    E = senders.shape[0]
    H = params["W0"].shape[1]
    D = edge_features.shape[1]

    s1 = senders.astype(jnp.int32)
    r1 = receivers.astype(jnp.int32)

    zeros_h = jnp.zeros((BODY, H), jnp.float32)
    zeros_hb = jnp.zeros((BODY, H), jnp.bfloat16)

    edge_call = _make_edge_kernel_bf16(N, E, H)

    ef32 = jnp.concatenate(
        [edge_features.astype(jnp.bfloat16),
         jnp.ones((E, 1), jnp.bfloat16),
         jnp.zeros((E, H - D - 1), jnp.bfloat16)], axis=-1)
    eidx = jnp.arange(E, dtype=jnp.int32)
    pre_p = edge_call(ef32, eidx, r1, zeros_hb)
    pre = pre_p[0].astype(jnp.float32) + pre_p[1].astype(jnp.float32)
    efs, deg = pre[:, :D], pre[:, D:D + 1]

    V = jnp.zeros_like(P_Q_inj).at[:, 0].set(1.0)
    h = P_Q_inj @ params["W0"] + params["b0"]
    g = jnp.zeros((1, H), jnp.float32)
    for lp in params["layers"]:
        Wm = lp["Wm"]
        nin = 2 + H
        T = jnp.concatenate([V, h], axis=-1) @ Wm[:nin]
        parts = edge_call(T.astype(jnp.bfloat16), s1, r1, zeros_hb)
        agg = parts[0].astype(jnp.float32) + parts[1].astype(jnp.float32)
        agg = agg + efs @ Wm[nin:] + deg * lp["bm"][None, :]
        h = jax.nn.relu(agg)
        nm = jnp.mean(h, axis=0, keepdims=True)
        g = jnp.concatenate([g, nm], axis=-1) @ lp["Wg"] + lp["bg"]
        h = jnp.concatenate([h, jnp.broadcast_to(g, (N, H))], axis=-1) @ lp["Wn"] + lp["bn"]
        V = V + h @ lp["Wd"] + lp["bd"]
    return V
